# initial kernel scaffold (unmeasured)
import jax
import jax.numpy as jnp
from jax import lax
from jax.experimental import pallas as pl
from jax.experimental.pallas import tpu as pltpu

N_DEV = 4
M_GLOBAL = 8192
D = 2048
M_CHUNK = M_GLOBAL // N_DEV


def kernel(partial, gamma):
    partial = partial.reshape(M_GLOBAL, D)
    gamma = gamma.reshape(1, D)

    def body(part_ref, gamma_ref, out_ref, comm_ref, stage_ref,
             copy_sem, send_sems, recv_sems):
        my = lax.axis_index("i")
        left = lax.rem(my + N_DEV - 1, N_DEV)
        right = lax.rem(my + 1, N_DEV)

        barrier_sem = pltpu.get_barrier_semaphore()
        for nbr in (left, right):
            pl.semaphore_signal(
                barrier_sem, inc=1,
                device_id=(nbr,), device_id_type=pl.DeviceIdType.MESH,
            )
        pl.semaphore_wait(barrier_sem, 2)

        def load_own_chunk(c):
            cp = pltpu.make_async_copy(
                part_ref.at[pl.ds(c * M_CHUNK, M_CHUNK), :],
                stage_ref,
                copy_sem,
            )
            cp.start()
            cp.wait()

        load_own_chunk(lax.rem(my + N_DEV - 1, N_DEV))

        for s in range(N_DEV - 1):
            src = stage_ref if s == 0 else comm_ref.at[s - 1]
            rdma = pltpu.make_async_remote_copy(
                src_ref=src,
                dst_ref=comm_ref.at[s],
                send_sem=send_sems.at[s],
                recv_sem=recv_sems.at[s],
                device_id=(right,),
                device_id_type=pl.DeviceIdType.MESH,
            )
            rdma.start()
            rdma.wait()
            c = lax.rem(my + 2 * N_DEV - 2 - s, N_DEV)
            load_own_chunk(c)
            comm_ref[s] = comm_ref[s] + stage_ref[...]

        y = comm_ref[N_DEV - 2]
        ms = jnp.mean(y * y, axis=-1, keepdims=True)
        out_ref[...] = y * lax.rsqrt(ms + 1e-6) * gamma_ref[...]

    return pl.pallas_call(
        body,
        out_shape=jax.ShapeDtypeStruct((M_CHUNK, D), jnp.float32),
        in_specs=[
            pl.BlockSpec(memory_space=pltpu.MemorySpace.ANY),
            pl.BlockSpec(memory_space=pltpu.VMEM),
        ],
        out_specs=pl.BlockSpec(memory_space=pltpu.VMEM),
        scratch_shapes=[
            pltpu.VMEM((N_DEV - 1, M_CHUNK, D), jnp.float32),
            pltpu.VMEM((M_CHUNK, D), jnp.float32),
            pltpu.SemaphoreType.DMA,
            pltpu.SemaphoreType.DMA((N_DEV - 1,)),
            pltpu.SemaphoreType.DMA((N_DEV - 1,)),
        ],
        compiler_params=pltpu.CompilerParams(
            collective_id=0,
            vmem_limit_bytes=128 * 1024 * 1024,
        ),
    )(partial, gamma)


# baseline (device time: 614855 ns/iter reference)
import jax
import jax.numpy as jnp
from jax import lax
from jax.experimental import pallas as pl
from jax.experimental.pallas import tpu as pltpu

N_DEV = 4
M_GLOBAL = 8192
D = 2048
M_CHUNK = M_GLOBAL // N_DEV
TILE = 512
T = M_CHUNK // TILE


def kernel(partial, gamma):
    partial = partial.reshape(M_GLOBAL, D)
    gamma = gamma.reshape(1, D)

    def body(part_ref, gamma_ref, out_ref, comm_ref, stage_ref,
             copy_sem, send_sems, recv_sems, credit_sem):
        my = lax.axis_index("i")
        left = lax.rem(my + N_DEV - 1, N_DEV)
        right = lax.rem(my + 1, N_DEV)

        barrier_sem = pltpu.get_barrier_semaphore()
        for nbr in (left, right):
            pl.semaphore_signal(
                barrier_sem, inc=1,
                device_id=(nbr,), device_id_type=pl.DeviceIdType.MESH,
            )
        pl.semaphore_wait(barrier_sem, 2)

        def load_own(c, t):
            cp = pltpu.make_async_copy(
                part_ref.at[pl.ds(c * M_CHUNK + t * TILE, TILE), :],
                stage_ref,
                copy_sem,
            )
            cp.start()
            cp.wait()

        for t in range(T):
            if t > 0:
                pl.semaphore_wait(credit_sem, 1)

            load_own(lax.rem(my + N_DEV - 1, N_DEV), t)

            for s in range(N_DEV - 1):
                src = stage_ref if s == 0 else comm_ref.at[s - 1]
                rdma = pltpu.make_async_remote_copy(
                    src_ref=src,
                    dst_ref=comm_ref.at[s],
                    send_sem=send_sems.at[s * T + t],
                    recv_sem=recv_sems.at[s * T + t],
                    device_id=(right,),
                    device_id_type=pl.DeviceIdType.MESH,
                )
                rdma.start()
                rdma.wait()
                c = lax.rem(my + 2 * N_DEV - 2 - s, N_DEV)
                load_own(c, t)
                if s < N_DEV - 2:
                    comm_ref[s] = comm_ref[s] + stage_ref[...]

            y = comm_ref[N_DEV - 2] + stage_ref[...]
            ms = jnp.mean(y * y, axis=-1, keepdims=True)
            out_ref[pl.ds(t * TILE, TILE), :] = (
                y * lax.rsqrt(ms + 1e-6) * gamma_ref[...]
            )

            if t < T - 1:
                pl.semaphore_signal(
                    credit_sem, inc=1,
                    device_id=(left,), device_id_type=pl.DeviceIdType.MESH,
                )

    return pl.pallas_call(
        body,
        out_shape=jax.ShapeDtypeStruct((M_CHUNK, D), jnp.float32),
        in_specs=[
            pl.BlockSpec(memory_space=pltpu.MemorySpace.HBM),
            pl.BlockSpec(memory_space=pltpu.VMEM),
        ],
        out_specs=pl.BlockSpec(memory_space=pltpu.VMEM),
        scratch_shapes=[
            pltpu.VMEM((N_DEV - 1, TILE, D), jnp.float32),
            pltpu.VMEM((TILE, D), jnp.float32),
            pltpu.SemaphoreType.DMA,
            pltpu.SemaphoreType.DMA(((N_DEV - 1) * T,)),
            pltpu.SemaphoreType.DMA(((N_DEV - 1) * T,)),
            pltpu.SemaphoreType.REGULAR,
        ],
        compiler_params=pltpu.CompilerParams(collective_id=0),
    )(partial, gamma)


# device time: 316727 ns/iter; 1.9413x vs baseline; 1.9413x over previous
import jax
import jax.numpy as jnp
from jax import lax
from jax.experimental import pallas as pl
from jax.experimental.pallas import tpu as pltpu

N_DEV = 4
M_GLOBAL = 8192
D = 2048
M_CHUNK = M_GLOBAL // N_DEV
TILE = 1024
T = M_CHUNK // TILE
HALF = TILE // 2


def kernel(partial, gamma):
    partial = partial.reshape(M_GLOBAL, D)
    gamma = gamma.reshape(1, D)

    def body(part_ref, gamma_ref, out_ref, commR, commL, stage, copy_sems,
             out_sems, sendR, recvR, sendL, recvL, credit_sem):
        my = lax.axis_index("i")
        left = lax.rem(my + N_DEV - 1, N_DEV)
        right = lax.rem(my + 1, N_DEV)

        barrier_sem = pltpu.get_barrier_semaphore()
        for nbr in (left, right):
            pl.semaphore_signal(
                barrier_sem, inc=1,
                device_id=(nbr,), device_id_type=pl.DeviceIdType.MESH,
            )
        pl.semaphore_wait(barrier_sem, 2)

        def stage_copy(d, slot, c, t):
            row0 = c * M_CHUNK + t * TILE + d * HALF
            return pltpu.make_async_copy(
                part_ref.at[pl.ds(row0, HALF), :],
                stage.at[d, slot],
                copy_sems.at[d * 2 + slot],
            )

        for t in range(T):
            if t > 0:
                pl.semaphore_wait(credit_sem, 2)

            cp0 = stage_copy(0, 0, lax.rem(my + N_DEV - 1, N_DEV), t)
            cp1 = stage_copy(1, 0, lax.rem(my + 1, N_DEV), t)
            cp0.start()
            cp1.start()
            cp0.wait()
            cp1.wait()

            for s in range(N_DEV - 1):
                srcR = stage.at[0, 0] if s == 0 else commR.at[s - 1]
                srcL = stage.at[1, 0] if s == 0 else commL.at[s - 1]
                rdmaR = pltpu.make_async_remote_copy(
                    src_ref=srcR,
                    dst_ref=commR.at[s],
                    send_sem=sendR.at[s * T + t],
                    recv_sem=recvR.at[s * T + t],
                    device_id=(right,),
                    device_id_type=pl.DeviceIdType.MESH,
                )
                rdmaL = pltpu.make_async_remote_copy(
                    src_ref=srcL,
                    dst_ref=commL.at[s],
                    send_sem=sendL.at[s * T + t],
                    recv_sem=recvL.at[s * T + t],
                    device_id=(left,),
                    device_id_type=pl.DeviceIdType.MESH,
                )
                rdmaR.start()
                rdmaL.start()

                slot = (s + 1) % 2
                cR = lax.rem(my + 2 * N_DEV - 2 - s, N_DEV)
                cL = lax.rem(my + s + 2, N_DEV)
                cpR = stage_copy(0, slot, cR, t)
                cpL = stage_copy(1, slot, cL, t)
                cpR.start()
                cpL.start()

                rdmaR.wait()
                rdmaL.wait()
                cpR.wait()
                cpL.wait()

                if s < N_DEV - 2:
                    commR[s] = commR[s] + stage[0, slot]
                    commL[s] = commL[s] + stage[1, slot]

            lastR = commR[N_DEV - 2] + stage[0, (N_DEV - 1) % 2]
            msR = jnp.mean(lastR * lastR, axis=-1, keepdims=True)
            commR[0] = lastR * lax.rsqrt(msR + 1e-6) * gamma_ref[...]
            lastL = commL[N_DEV - 2] + stage[1, (N_DEV - 1) % 2]
            msL = jnp.mean(lastL * lastL, axis=-1, keepdims=True)
            commL[0] = lastL * lax.rsqrt(msL + 1e-6) * gamma_ref[...]
            outR = pltpu.make_async_copy(
                commR.at[0], out_ref.at[pl.ds(t * TILE, HALF), :],
                out_sems.at[0],
            )
            outL = pltpu.make_async_copy(
                commL.at[0], out_ref.at[pl.ds(t * TILE + HALF, HALF), :],
                out_sems.at[1],
            )
            outR.start()
            outL.start()
            outR.wait()
            outL.wait()

            if t < T - 1:
                for nbr in (left, right):
                    pl.semaphore_signal(
                        credit_sem, inc=1,
                        device_id=(nbr,), device_id_type=pl.DeviceIdType.MESH,
                    )

    n_sems = (N_DEV - 1) * T
    return pl.pallas_call(
        body,
        out_shape=jax.ShapeDtypeStruct((M_CHUNK, D), jnp.float32),
        in_specs=[
            pl.BlockSpec(memory_space=pltpu.MemorySpace.HBM),
            pl.BlockSpec(memory_space=pltpu.MemorySpace.VMEM),
        ],
        out_specs=pl.BlockSpec(memory_space=pltpu.MemorySpace.HBM),
        scratch_shapes=[
            pltpu.VMEM((N_DEV - 1, HALF, D), jnp.float32),
            pltpu.VMEM((N_DEV - 1, HALF, D), jnp.float32),
            pltpu.VMEM((2, 2, HALF, D), jnp.float32),
            pltpu.SemaphoreType.DMA((4,)),
            pltpu.SemaphoreType.DMA((2,)),
            pltpu.SemaphoreType.DMA((n_sems,)),
            pltpu.SemaphoreType.DMA((n_sems,)),
            pltpu.SemaphoreType.DMA((n_sems,)),
            pltpu.SemaphoreType.DMA((n_sems,)),
            pltpu.SemaphoreType.REGULAR,
        ],
        compiler_params=pltpu.CompilerParams(
            collective_id=0,
            vmem_limit_bytes=60 * 1024 * 1024,
        ),
    )(partial, gamma)


# device time: 293777 ns/iter; 2.0929x vs baseline; 1.0781x over previous
import jax
import jax.numpy as jnp
from jax import lax
from jax.experimental import pallas as pl
from jax.experimental.pallas import tpu as pltpu

N_DEV = 4
M_GLOBAL = 8192
D = 2048
M_CHUNK = M_GLOBAL // N_DEV
TILE = 1024
T = M_CHUNK // TILE
HALF = TILE // 2
N_HOP = N_DEV - 1


def kernel(partial, gamma):
    partial = partial.reshape(M_GLOBAL, D)
    gamma = gamma.reshape(1, D)

    def body(part_ref, gamma_ref, out_ref, commR, commL, stage,
             copy_sems, out_sems, sendR, recvR, sendL, recvL):
        my = lax.axis_index("i")
        left = lax.rem(my + N_DEV - 1, N_DEV)
        right = lax.rem(my + 1, N_DEV)

        barrier_sem = pltpu.get_barrier_semaphore()
        for nbr in (left, right):
            pl.semaphore_signal(
                barrier_sem, inc=1,
                device_id=(nbr,), device_id_type=pl.DeviceIdType.MESH,
            )
        pl.semaphore_wait(barrier_sem, 2)

        def own_rows(d, c, t):
            row0 = c * M_CHUNK + t * TILE + d * HALF
            return part_ref.at[pl.ds(row0, HALF), :]

        def make_rdma(d, t, s):
            comm = commR if d == 0 else commL
            if s == 0:
                c = lax.rem(my + N_DEV - 1, N_DEV) if d == 0 \
                    else lax.rem(my + 1, N_DEV)
                src = own_rows(d, c, t)
            else:
                src = comm.at[t, (s - 1) % 2]
            send = (sendR if d == 0 else sendL).at[s * T + t]
            recv = (recvR if d == 0 else recvL).at[s * T + t]
            return pltpu.make_async_remote_copy(
                src_ref=src,
                dst_ref=comm.at[t, s % 2],
                send_sem=send,
                recv_sem=recv,
                device_id=(right,) if d == 0 else (left,),
                device_id_type=pl.DeviceIdType.MESH,
            )

        def make_prefetch(d, t, s):
            c = lax.rem(my + 2 * N_DEV - 2 - s, N_DEV) if d == 0 \
                else lax.rem(my + s + 2, N_DEV)
            return pltpu.make_async_copy(
                own_rows(d, c, t),
                stage.at[d, t],
                copy_sems.at[d * T + t],
            )

        rdmas = {}
        prefetch = {}
        for t in range(T):
            for d in range(2):
                rdmas[d, t] = make_rdma(d, t, 0)
                rdmas[d, t].start()
        for t in range(T):
            for d in range(2):
                prefetch[d, t] = make_prefetch(d, t, 0)
                prefetch[d, t].start()

        out_copies = []
        for s in range(N_HOP):
            for t in range(T):
                for d in range(2):
                    rdmas[d, t].wait()
                    prefetch[d, t].wait()
                comm = (commR, commL)
                if s < N_HOP - 1:
                    for d in range(2):
                        comm[d][t, s % 2] = (
                            comm[d][t, s % 2] + stage[d, t]
                        )
                        rdmas[d, t] = make_rdma(d, t, s + 1)
                        rdmas[d, t].start()
                        prefetch[d, t] = make_prefetch(d, t, s + 1)
                        prefetch[d, t].start()
                else:
                    for d in range(2):
                        y = comm[d][t, (N_HOP - 1) % 2] + stage[d, t]
                        ms = jnp.mean(y * y, axis=-1, keepdims=True)
                        comm[d][t, (N_HOP - 2) % 2] = (
                            y * lax.rsqrt(ms + 1e-6) * gamma_ref[...]
                        )
                        oc = pltpu.make_async_copy(
                            comm[d].at[t, (N_HOP - 2) % 2],
                            out_ref.at[pl.ds(t * TILE + d * HALF, HALF), :],
                            out_sems.at[d * T + t],
                        )
                        oc.start()
                        out_copies.append(oc)
        for oc in out_copies:
            oc.wait()

    n_sems = N_HOP * T
    return pl.pallas_call(
        body,
        out_shape=jax.ShapeDtypeStruct((M_CHUNK, D), jnp.float32),
        in_specs=[
            pl.BlockSpec(memory_space=pltpu.MemorySpace.HBM),
            pl.BlockSpec(memory_space=pltpu.MemorySpace.VMEM),
        ],
        out_specs=pl.BlockSpec(memory_space=pltpu.MemorySpace.HBM),
        scratch_shapes=[
            pltpu.VMEM((T, 2, HALF, D), jnp.float32),
            pltpu.VMEM((T, 2, HALF, D), jnp.float32),
            pltpu.VMEM((2, T, HALF, D), jnp.float32),
            pltpu.SemaphoreType.DMA((2 * T,)),
            pltpu.SemaphoreType.DMA((2 * T,)),
            pltpu.SemaphoreType.DMA((n_sems,)),
            pltpu.SemaphoreType.DMA((n_sems,)),
            pltpu.SemaphoreType.DMA((n_sems,)),
            pltpu.SemaphoreType.DMA((n_sems,)),
        ],
        compiler_params=pltpu.CompilerParams(
            collective_id=0,
            vmem_limit_bytes=60 * 1024 * 1024,
        ),
    )(partial, gamma)


# device time: 291968 ns/iter; 2.1059x vs baseline; 1.0062x over previous
import jax
import jax.numpy as jnp
from jax import lax
from jax.experimental import pallas as pl
from jax.experimental.pallas import tpu as pltpu

N_DEV = 4
M_GLOBAL = 8192
D = 2048
M_CHUNK = M_GLOBAL // N_DEV
TILE = 512
T = M_CHUNK // TILE
HALF = TILE // 2
N_HOP = N_DEV - 1


def kernel(partial, gamma):
    partial = partial.reshape(M_GLOBAL, D)
    gamma = gamma.reshape(1, D)

    def body(part_ref, gamma_ref, out_ref, commR, commL, stage,
             copy_sems, out_sems, sendR, recvR, sendL, recvL):
        my = lax.axis_index("i")
        left = lax.rem(my + N_DEV - 1, N_DEV)
        right = lax.rem(my + 1, N_DEV)

        barrier_sem = pltpu.get_barrier_semaphore()
        for nbr in (left, right):
            pl.semaphore_signal(
                barrier_sem, inc=1,
                device_id=(nbr,), device_id_type=pl.DeviceIdType.MESH,
            )
        pl.semaphore_wait(barrier_sem, 2)

        def own_rows(d, c, t):
            row0 = c * M_CHUNK + t * TILE + d * HALF
            return part_ref.at[pl.ds(row0, HALF), :]

        def make_rdma(d, t, s):
            comm = commR if d == 0 else commL
            if s == 0:
                c = lax.rem(my + N_DEV - 1, N_DEV) if d == 0 \
                    else lax.rem(my + 1, N_DEV)
                src = own_rows(d, c, t)
            else:
                src = comm.at[t, (s - 1) % 2]
            send = (sendR if d == 0 else sendL).at[s * T + t]
            recv = (recvR if d == 0 else recvL).at[s * T + t]
            return pltpu.make_async_remote_copy(
                src_ref=src,
                dst_ref=comm.at[t, s % 2],
                send_sem=send,
                recv_sem=recv,
                device_id=(right,) if d == 0 else (left,),
                device_id_type=pl.DeviceIdType.MESH,
            )

        def make_prefetch(d, t, s):
            c = lax.rem(my + 2 * N_DEV - 2 - s, N_DEV) if d == 0 \
                else lax.rem(my + s + 2, N_DEV)
            return pltpu.make_async_copy(
                own_rows(d, c, t),
                stage.at[d, t],
                copy_sems.at[d * T + t],
            )

        rdmas = {}
        prefetch = {}
        for t in range(T):
            for d in range(2):
                rdmas[d, t] = make_rdma(d, t, 0)
                rdmas[d, t].start()
        for t in range(T):
            for d in range(2):
                prefetch[d, t] = make_prefetch(d, t, 0)
                prefetch[d, t].start()

        out_copies = []
        for s in range(N_HOP):
            for t in range(T):
                for d in range(2):
                    rdmas[d, t].wait()
                    prefetch[d, t].wait()
                comm = (commR, commL)
                if s < N_HOP - 1:
                    for d in range(2):
                        comm[d][t, s % 2] = (
                            comm[d][t, s % 2] + stage[d, t]
                        )
                        rdmas[d, t] = make_rdma(d, t, s + 1)
                        rdmas[d, t].start()
                        prefetch[d, t] = make_prefetch(d, t, s + 1)
                        prefetch[d, t].start()
                else:
                    for d in range(2):
                        y = comm[d][t, (N_HOP - 1) % 2] + stage[d, t]
                        ms = jnp.mean(y * y, axis=-1, keepdims=True)
                        comm[d][t, (N_HOP - 2) % 2] = (
                            y * lax.rsqrt(ms + 1e-6) * gamma_ref[...]
                        )
                        oc = pltpu.make_async_copy(
                            comm[d].at[t, (N_HOP - 2) % 2],
                            out_ref.at[pl.ds(t * TILE + d * HALF, HALF), :],
                            out_sems.at[d * T + t],
                        )
                        oc.start()
                        out_copies.append(oc)
        for oc in out_copies:
            oc.wait()

    n_sems = N_HOP * T
    return pl.pallas_call(
        body,
        out_shape=jax.ShapeDtypeStruct((M_CHUNK, D), jnp.float32),
        in_specs=[
            pl.BlockSpec(memory_space=pltpu.MemorySpace.HBM),
            pl.BlockSpec(memory_space=pltpu.MemorySpace.VMEM),
        ],
        out_specs=pl.BlockSpec(memory_space=pltpu.MemorySpace.HBM),
        scratch_shapes=[
            pltpu.VMEM((T, 2, HALF, D), jnp.float32),
            pltpu.VMEM((T, 2, HALF, D), jnp.float32),
            pltpu.VMEM((2, T, HALF, D), jnp.float32),
            pltpu.SemaphoreType.DMA((2 * T,)),
            pltpu.SemaphoreType.DMA((2 * T,)),
            pltpu.SemaphoreType.DMA((n_sems,)),
            pltpu.SemaphoreType.DMA((n_sems,)),
            pltpu.SemaphoreType.DMA((n_sems,)),
            pltpu.SemaphoreType.DMA((n_sems,)),
        ],
        compiler_params=pltpu.CompilerParams(
            collective_id=0,
            vmem_limit_bytes=60 * 1024 * 1024,
        ),
    )(partial, gamma)


# device time: 290816 ns/iter; 2.1142x vs baseline; 1.0040x over previous
import jax
import jax.numpy as jnp
from jax import lax
from jax.experimental import pallas as pl
from jax.experimental.pallas import tpu as pltpu

N_DEV = 4
M_GLOBAL = 8192
D = 2048
M_CHUNK = M_GLOBAL // N_DEV
TILE = 512
T = M_CHUNK // TILE
HALF = TILE // 2
N_HOP = N_DEV - 1


def kernel(partial, gamma):

    def body(part_ref, gamma_ref, out_ref, commR, commL, stage,
             copy_sems, out_sems, sendR, recvR, sendL, recvL):
        my = lax.axis_index("i")
        left = lax.rem(my + N_DEV - 1, N_DEV)
        right = lax.rem(my + 1, N_DEV)

        barrier_sem = pltpu.get_barrier_semaphore()
        for nbr in (left, right):
            pl.semaphore_signal(
                barrier_sem, inc=1,
                device_id=(nbr,), device_id_type=pl.DeviceIdType.MESH,
            )
        pl.semaphore_wait(barrier_sem, 2)

        def own_rows(d, c, t):
            row0 = c * M_CHUNK + t * TILE + d * HALF
            return part_ref.at[0, pl.ds(row0, HALF), :]

        def make_rdma(d, t, s):
            comm = commR if d == 0 else commL
            if s == 0:
                c = lax.rem(my + N_DEV - 1, N_DEV) if d == 0 \
                    else lax.rem(my + 1, N_DEV)
                src = own_rows(d, c, t)
            else:
                src = comm.at[t, (s - 1) % 2]
            send = (sendR if d == 0 else sendL).at[s * T + t]
            recv = (recvR if d == 0 else recvL).at[s * T + t]
            return pltpu.make_async_remote_copy(
                src_ref=src,
                dst_ref=comm.at[t, s % 2],
                send_sem=send,
                recv_sem=recv,
                device_id=(right,) if d == 0 else (left,),
                device_id_type=pl.DeviceIdType.MESH,
            )

        def make_prefetch(d, t, s):
            c = lax.rem(my + 2 * N_DEV - 2 - s, N_DEV) if d == 0 \
                else lax.rem(my + s + 2, N_DEV)
            return pltpu.make_async_copy(
                own_rows(d, c, t),
                stage.at[d, t],
                copy_sems.at[d * T + t],
            )

        rdmas = {}
        prefetch = {}
        for t in range(T):
            for d in range(2):
                rdmas[d, t] = make_rdma(d, t, 0)
                rdmas[d, t].start()
        for t in range(T):
            for d in range(2):
                prefetch[d, t] = make_prefetch(d, t, 0)
                prefetch[d, t].start()

        out_copies = []
        for s in range(N_HOP):
            for t in range(T):
                for d in range(2):
                    rdmas[d, t].wait()
                    prefetch[d, t].wait()
                comm = (commR, commL)
                if s < N_HOP - 1:
                    for d in range(2):
                        comm[d][t, s % 2] = (
                            comm[d][t, s % 2] + stage[d, t]
                        )
                        rdmas[d, t] = make_rdma(d, t, s + 1)
                        rdmas[d, t].start()
                        prefetch[d, t] = make_prefetch(d, t, s + 1)
                        prefetch[d, t].start()
                else:
                    for d in range(2):
                        y = comm[d][t, (N_HOP - 1) % 2] + stage[d, t]
                        ms = jnp.mean(y * y, axis=-1, keepdims=True)
                        comm[d][t, (N_HOP - 2) % 2] = (
                            y * lax.rsqrt(ms + 1e-6) * gamma_ref[...][None, :]
                        )
                        oc = pltpu.make_async_copy(
                            comm[d].at[t, (N_HOP - 2) % 2],
                            out_ref.at[pl.ds(t * TILE + d * HALF, HALF), :],
                            out_sems.at[d * T + t],
                        )
                        oc.start()
                        out_copies.append(oc)
        for oc in out_copies:
            oc.wait()

    n_sems = N_HOP * T
    return pl.pallas_call(
        body,
        out_shape=jax.ShapeDtypeStruct((M_CHUNK, D), jnp.float32),
        in_specs=[
            pl.BlockSpec(memory_space=pltpu.MemorySpace.HBM),
            pl.BlockSpec(memory_space=pltpu.MemorySpace.VMEM),
        ],
        out_specs=pl.BlockSpec(memory_space=pltpu.MemorySpace.HBM),
        scratch_shapes=[
            pltpu.VMEM((T, 2, HALF, D), jnp.float32),
            pltpu.VMEM((T, 2, HALF, D), jnp.float32),
            pltpu.VMEM((2, T, HALF, D), jnp.float32),
            pltpu.SemaphoreType.DMA((2 * T,)),
            pltpu.SemaphoreType.DMA((2 * T,)),
            pltpu.SemaphoreType.DMA((n_sems,)),
            pltpu.SemaphoreType.DMA((n_sems,)),
            pltpu.SemaphoreType.DMA((n_sems,)),
            pltpu.SemaphoreType.DMA((n_sems,)),
        ],
        compiler_params=pltpu.CompilerParams(
            collective_id=0,
            vmem_limit_bytes=60 * 1024 * 1024,
        ),
    )(partial, gamma)
